# baseline (speedup 1.0000x reference)
import functools

import jax
import jax.numpy as jnp
from jax.experimental import pallas as pl
from jax.experimental.pallas import tpu as pltpu

_B, _N, _F, _H = 2, 512, 128, 64
_BI = 64


def _silu(x):
    return x * jax.nn.sigmoid(x)


def _bdot(a, b):
    return jnp.dot(a.astype(jnp.bfloat16), b.astype(jnp.bfloat16),
                   preferred_element_type=jnp.float32)


def _egnn_kernel(nf_ref, pos_ref, posT_ref, valid_ref, adj_ref, Wemb_ref,
                 We1a0_ref, We1b0_ref, we1c0_ref, We2_0_ref, Wx1_0_ref, Wx2_0_ref,
                 Wh1_0_ref, Wh2_0_ref,
                 We1a1_ref, We1b1_ref, we1c1_ref, We2_1_ref,
                 Wh1_1_ref, Wh2_1_ref,
                 Wp1a_ref, Wp1b_ref, Wp2a_ref, Wp2b_ref,
                 out_ref,
                 h_s, pos_s, mask_s, A_s, agg_s, dp_s):
    f32 = jnp.float32
    hp = jax.lax.Precision.HIGHEST
    vf = valid_ref[0].astype(f32)
    mask_s[...] = adj_ref[0].astype(f32) * vf * vf.reshape(_N, 1)
    h_s[...] = _bdot(nf_ref[0], Wemb_ref[...])
    pos_s[...] = pos_ref[0]
    posT = posT_ref[0]

    layer_ws = [
        (We1a0_ref, We1b0_ref, we1c0_ref, We2_0_ref, Wx1_0_ref, Wx2_0_ref,
         Wh1_0_ref, Wh2_0_ref),
        (We1a1_ref, We1b1_ref, we1c1_ref, We2_1_ref, None, None,
         Wh1_1_ref, Wh2_1_ref),
    ]

    for l, (We1a, We1b, we1c, We2, Wx1, Wx2, Wh1, Wh2) in enumerate(layer_ws):
        h = h_s[...]
        pos = pos_s[...]
        A_s[...] = _bdot(h, We1a[...])
        C = _bdot(h, We1b[...])
        n2_row = jnp.sum(pos * pos, axis=1).reshape(1, _N)
        w1c = we1c[...].reshape(_H).astype(jnp.bfloat16).astype(f32)

        def body(ib, carry, We2=We2, Wx1=Wx1, Wx2=Wx2, C=C,
                 n2_row=n2_row, w1c=w1c, l=l):
            sl = pl.ds(ib * _BI, _BI)
            posb = pos_s[sl, :]
            if l == 0:
                rx = posb[:, 0:1] - posT[0:1, :]
                ry = posb[:, 1:2] - posT[1:2, :]
                rz = posb[:, 2:3] - posT[2:3, :]
                d2 = rx * rx + ry * ry + rz * rz
            else:
                gram = jax.lax.dot_general(
                    posb, pos_s[...], (((1,), (1,)), ((), ())),
                    precision=hp, preferred_element_type=f32)
                n2b = jnp.sum(posb * posb, axis=1)
                d2 = n2b[:, None] + n2_row - 2.0 * gram
            d2b = d2.astype(jnp.bfloat16).astype(f32)
            pre = (A_s[sl, :][:, None, :]
                   + (C[None, :, :]
                      + d2b[:, :, None] * w1c[None, None, :]))
            s = _silu(pre).reshape(_BI * _N, _H)
            m = _silu(_bdot(s, We2[...]))
            m3 = m.reshape(_BI, _N, _H) * mask_s[sl, :][:, :, None]
            agg_s[sl, :] = jnp.sum(m3, axis=1)
            if l == 0:
                u = _silu(_bdot(m3.reshape(_BI * _N, _H), Wx1[...]))
                ub = u.reshape(_BI, _N, _H).astype(jnp.bfloat16).astype(f32)
                wx2 = Wx2[...].reshape(_H).astype(jnp.bfloat16).astype(f32)
                coef = jnp.sum(ub * wx2[None, None, :], axis=2)
                dx = jnp.sum(coef * rx, axis=1, keepdims=True)
                dy = jnp.sum(coef * ry, axis=1, keepdims=True)
                dz = jnp.sum(coef * rz, axis=1, keepdims=True)
                dp_s[sl, :] = jnp.concatenate([dx, dy, dz],
                                              axis=1) / (_N - 1)
            return carry

        jax.lax.fori_loop(0, _N // _BI, body, 0)

        hid = _silu(_bdot(h, Wh1[0:_H, :]) + _bdot(agg_s[...], Wh1[_H:, :]))
        h_s[...] = h + _bdot(hid, Wh2[...])
        if l == 0:
            pos_s[...] = pos + dp_s[...]

    h = h_s[...]
    p = _bdot(_silu(_bdot(h, Wp1a_ref[...])), Wp1b_ref[...])
    ps = jnp.sum(p, axis=0, keepdims=True)
    out = _bdot(_silu(_bdot(ps, Wp2a_ref[...])), Wp2b_ref[...])
    out_ref[...] = out.reshape(1, 1, 1)


@functools.partial(jax.jit, static_argnames=("interpret",))
def _run(node_feat, pos, valid, adj, W_embed,
         We1_0, We2_0, Wx1_0, Wx2_0, Wh1_0, Wh2_0,
         We1_1, We2_1, Wx1_1, Wx2_1, Wh1_1, Wh2_1,
         Wp1a, Wp1b, Wp2a, Wp2b, interpret=False):
    H = _H
    args = (
        node_feat, pos, pos.transpose(0, 2, 1), valid.reshape(_B, 1, _N), adj,
        W_embed,
        We1_0[:H], We1_0[H:2 * H], We1_0[2 * H:], We2_0, Wx1_0, Wx2_0,
        Wh1_0, Wh2_0,
        We1_1[:H], We1_1[H:2 * H], We1_1[2 * H:], We2_1,
        Wh1_1, Wh2_1,
        Wp1a, Wp1b, Wp2a, Wp2b,
    )
    batch_specs = [
        pl.BlockSpec((1, _N, _F), lambda b: (b, 0, 0)),
        pl.BlockSpec((1, _N, 3), lambda b: (b, 0, 0)),
        pl.BlockSpec((1, 3, _N), lambda b: (b, 0, 0)),
        pl.BlockSpec((1, 1, _N), lambda b: (b, 0, 0)),
        pl.BlockSpec((1, _N, _N), lambda b: (b, 0, 0)),
    ]
    weight_specs = [pl.BlockSpec(a.shape, lambda b: (0,) * a.ndim)
                    for a in args[5:]]
    scratch = [
        pltpu.VMEM((_N, _H), jnp.float32),
        pltpu.VMEM((_N, 3), jnp.float32),
        pltpu.VMEM((_N, _N), jnp.float32),
        pltpu.VMEM((_N, _H), jnp.float32),
        pltpu.VMEM((_N, _H), jnp.float32),
        pltpu.VMEM((_N, 3), jnp.float32),
    ]
    out = pl.pallas_call(
        _egnn_kernel,
        grid=(_B,),
        in_specs=batch_specs + weight_specs,
        out_specs=pl.BlockSpec((1, 1, 1), lambda b: (b, 0, 0)),
        out_shape=jax.ShapeDtypeStruct((_B, 1, 1), jnp.float32),
        scratch_shapes=scratch,
        compiler_params=pltpu.CompilerParams(
            dimension_semantics=("parallel",)),
        interpret=interpret,
    )(*args)
    return out.reshape(_B, 1)


def kernel(node_feat, pos, valid, adj, W_embed,
           We1_0, We2_0, Wx1_0, Wx2_0, Wh1_0, Wh2_0,
           We1_1, We2_1, Wx1_1, Wx2_1, Wh1_1, Wh2_1,
           Wp1a, Wp1b, Wp2a, Wp2b):
    return _run(node_feat, pos, valid, adj, W_embed,
                We1_0, We2_0, Wx1_0, Wx2_0, Wh1_0, Wh2_0,
                We1_1, We2_1, Wx1_1, Wx2_1, Wh1_1, Wh2_1,
                Wp1a, Wp1b, Wp2a, Wp2b)


# speedup vs baseline: 1.8716x; 1.8716x over previous
import functools

import jax
import jax.numpy as jnp
from jax.experimental import pallas as pl
from jax.experimental.pallas import tpu as pltpu

_B, _N, _F, _H = 2, 512, 128, 64
_BI = 64


def _silu(x):
    return x * jax.nn.sigmoid(x)


def _bdot(a, b):
    return jnp.dot(a.astype(jnp.bfloat16), b.astype(jnp.bfloat16),
                   preferred_element_type=jnp.float32)


def _egnn_kernel(nf_ref, pos_ref, posT_ref, valid_ref, adj_ref, Wemb_ref,
                 We1a0_ref, We1b0_ref, we1c0_ref, We2_0_ref, Wx1_0_ref, Wx2_0_ref,
                 Wh1_0_ref, Wh2_0_ref,
                 We1a1_ref, We1b1_ref, we1c1_ref, We2_1_ref,
                 Wh1_1_ref, Wh2_1_ref,
                 Wp1a_ref, Wp1b_ref, Wp2a_ref, Wp2b_ref,
                 out_ref,
                 h_s, pos_s, mask_s, A_s, agg_s, dp_s):
    f32 = jnp.float32
    hp = jax.lax.Precision.HIGHEST
    vf = valid_ref[0].astype(f32)
    mask_s[...] = adj_ref[0].astype(f32) * vf * vf.reshape(_N, 1)
    h_s[...] = _bdot(nf_ref[0], Wemb_ref[...])
    pos_s[...] = pos_ref[0]
    posT = posT_ref[0]

    layer_ws = [
        (We1a0_ref, We1b0_ref, we1c0_ref, We2_0_ref, Wx1_0_ref, Wx2_0_ref,
         Wh1_0_ref, Wh2_0_ref),
        (We1a1_ref, We1b1_ref, we1c1_ref, We2_1_ref, None, None,
         Wh1_1_ref, Wh2_1_ref),
    ]

    for l, (We1a, We1b, we1c, We2, Wx1, Wx2, Wh1, Wh2) in enumerate(layer_ws):
        h = h_s[...]
        pos = pos_s[...]
        A_s[...] = _bdot(h, We1a[...])
        C = _bdot(h, We1b[...])
        n2_row = jnp.sum(pos * pos, axis=1).reshape(1, _N)
        w1c = we1c[...].reshape(_H).astype(jnp.bfloat16).astype(f32)

        def body(ib, carry, We2=We2, Wx1=Wx1, Wx2=Wx2, C=C,
                 n2_row=n2_row, w1c=w1c, l=l):
            sl = pl.ds(ib * _BI, _BI)
            posb = pos_s[sl, :]
            gram = jax.lax.dot_general(
                posb, pos_s[...], (((1,), (1,)), ((), ())),
                precision=hp, preferred_element_type=f32)
            n2b = jnp.sum(posb * posb, axis=1)
            d2 = n2b[:, None] + n2_row - 2.0 * gram
            d2b = d2.astype(jnp.bfloat16).astype(f32)
            pre = (A_s[sl, :][:, None, :]
                   + (C[None, :, :]
                      + d2b[:, :, None] * w1c[None, None, :]))
            s = _silu(pre).reshape(_BI * _N, _H)
            m = _silu(_bdot(s, We2[...]))
            m3 = m.reshape(_BI, _N, _H) * mask_s[sl, :][:, :, None]
            agg_s[sl, :] = jnp.sum(m3, axis=1)
            if l == 0:
                u = _silu(_bdot(m3.reshape(_BI * _N, _H), Wx1[...]))
                ub = u.reshape(_BI, _N, _H).astype(jnp.bfloat16).astype(f32)
                wx2 = Wx2[...].reshape(_H).astype(jnp.bfloat16).astype(f32)
                coef = jnp.sum(ub * wx2[None, None, :], axis=2)
                rw = jnp.sum(coef, axis=1)
                dp_s[sl, :] = (rw[:, None] * posb
                               - jnp.dot(coef, pos_s[...], precision=hp,
                                         preferred_element_type=f32)) / (_N - 1)
            return carry

        jax.lax.fori_loop(0, _N // _BI, body, 0)

        hid = _silu(_bdot(h, Wh1[0:_H, :]) + _bdot(agg_s[...], Wh1[_H:, :]))
        h_s[...] = h + _bdot(hid, Wh2[...])
        if l == 0:
            pos_s[...] = pos + dp_s[...]

    h = h_s[...]
    p = _bdot(_silu(_bdot(h, Wp1a_ref[...])), Wp1b_ref[...])
    ps = jnp.sum(p, axis=0, keepdims=True)
    out = _bdot(_silu(_bdot(ps, Wp2a_ref[...])), Wp2b_ref[...])
    out_ref[...] = out.reshape(1, 1, 1)


@functools.partial(jax.jit, static_argnames=("interpret",))
def _run(node_feat, pos, valid, adj, W_embed,
         We1_0, We2_0, Wx1_0, Wx2_0, Wh1_0, Wh2_0,
         We1_1, We2_1, Wx1_1, Wx2_1, Wh1_1, Wh2_1,
         Wp1a, Wp1b, Wp2a, Wp2b, interpret=False):
    H = _H
    args = (
        node_feat, pos, pos.transpose(0, 2, 1), valid.reshape(_B, 1, _N), adj,
        W_embed,
        We1_0[:H], We1_0[H:2 * H], We1_0[2 * H:], We2_0, Wx1_0, Wx2_0,
        Wh1_0, Wh2_0,
        We1_1[:H], We1_1[H:2 * H], We1_1[2 * H:], We2_1,
        Wh1_1, Wh2_1,
        Wp1a, Wp1b, Wp2a, Wp2b,
    )
    batch_specs = [
        pl.BlockSpec((1, _N, _F), lambda b: (b, 0, 0)),
        pl.BlockSpec((1, _N, 3), lambda b: (b, 0, 0)),
        pl.BlockSpec((1, 3, _N), lambda b: (b, 0, 0)),
        pl.BlockSpec((1, 1, _N), lambda b: (b, 0, 0)),
        pl.BlockSpec((1, _N, _N), lambda b: (b, 0, 0)),
    ]
    weight_specs = [pl.BlockSpec(a.shape, lambda b: (0,) * a.ndim)
                    for a in args[5:]]
    scratch = [
        pltpu.VMEM((_N, _H), jnp.float32),
        pltpu.VMEM((_N, 3), jnp.float32),
        pltpu.VMEM((_N, _N), jnp.float32),
        pltpu.VMEM((_N, _H), jnp.float32),
        pltpu.VMEM((_N, _H), jnp.float32),
        pltpu.VMEM((_N, 3), jnp.float32),
    ]
    out = pl.pallas_call(
        _egnn_kernel,
        grid=(_B,),
        in_specs=batch_specs + weight_specs,
        out_specs=pl.BlockSpec((1, 1, 1), lambda b: (b, 0, 0)),
        out_shape=jax.ShapeDtypeStruct((_B, 1, 1), jnp.float32),
        scratch_shapes=scratch,
        compiler_params=pltpu.CompilerParams(
            dimension_semantics=("parallel",)),
        interpret=interpret,
    )(*args)
    return out.reshape(_B, 1)


def kernel(node_feat, pos, valid, adj, W_embed,
           We1_0, We2_0, Wx1_0, Wx2_0, Wh1_0, Wh2_0,
           We1_1, We2_1, Wx1_1, Wx2_1, Wh1_1, Wh2_1,
           Wp1a, Wp1b, Wp2a, Wp2b):
    return _run(node_feat, pos, valid, adj, W_embed,
                We1_0, We2_0, Wx1_0, Wx2_0, Wh1_0, Wh2_0,
                We1_1, We2_1, Wx1_1, Wx2_1, Wh1_1, Wh2_1,
                Wp1a, Wp1b, Wp2a, Wp2b)
